# Initial kernel scaffold; baseline (speedup 1.0000x reference)
import jax
import jax.numpy as jnp
from jax.experimental import pallas as pl


def kernel(f, pos, edge_attr, targets, edge_index, params):
    raise NotImplementedError("write your pallas kernel here")


# speedup vs baseline: 4.2533x; 4.2533x over previous
import functools
from typing import Any

import jax
import jax.numpy as jnp
import numpy as np
from jax.experimental import pallas as pl
from jax.experimental.pallas import tpu as pltpu

N = 10000
E = 640000
EDGE_DIM = 4
NUM_LAYERS = 4
N_HEADS = 8
C0, C1 = 32, 16
RH = 32
HD = C0 // N_HEADS

BE = 2000


def _edge_kernel(ec_ref, f0_ref, f1_ref,
                 wr1_ref, br1_ref, wr2_ref, br2_ref,
                 w00_ref, w10_ref, w01_ref, w11_ref,
                 m0_ref, m1_ref):
    ec = ec_ref[...]
    f0 = f0_ref[...]
    f1 = f1_ref[...]

    dot = lambda a, b: jnp.dot(a, b, preferred_element_type=jnp.float32)

    rad = ec
    hwr = jnp.maximum(dot(rad, wr1_ref[...]) + br1_ref[...], 0.0)
    w = dot(hwr, wr2_ref[...]) + br2_ref[...]
    w0 = w[:, :C0]
    w1 = w[:, C0:C0 + C1]

    d0 = ec[:, 5:6]
    d1 = ec[:, 6:7]
    d2 = ec[:, 7:8]
    f1a = f1[:, 0:16]
    f1b = f1[:, 16:32]
    f1c = f1[:, 32:48]
    bf = lambda x: x.astype(jnp.bfloat16).astype(jnp.float32)
    f1d = bf(f1a) * bf(d0) + bf(f1b) * bf(d1) + bf(f1c) * bf(d2)

    m0 = (dot(f0, w00_ref[...]) + dot(f1d, w10_ref[...])) * w0
    g = dot(f0, w01_ref[...])
    w11 = w11_ref[...]
    m1a = (dot(f1a, w11) + g * d0) * w1
    m1b = (dot(f1b, w11) + g * d1) * w1
    m1c = (dot(f1c, w11) + g * d2) * w1

    m0_ref[...] = m0
    m1_ref[...] = jnp.concatenate([m1a, m1b, m1c], axis=1)


def _run_edge_layer(ec, f0s, f1s, lp):
    wr1 = jnp.zeros((8, RH), jnp.float32).at[:5, :].set(lp["Wr1"])
    wr2 = lp["Wr2"][:, :C0 + C1]
    br2 = lp["br2"][:C0 + C1]

    grid = (E // BE,)
    eb = lambda w: pl.BlockSpec((BE, w), lambda i: (i, 0))
    full = lambda a: pl.BlockSpec(a.shape, lambda i: (0,) * a.ndim)
    args = (ec, f0s, f1s, wr1, lp["br1"], wr2, br2, lp["W00"], lp["W10"],
            lp["W01"], lp["W11"])
    return pl.pallas_call(
        _edge_kernel,
        grid=grid,
        in_specs=[eb(8), eb(32), eb(48)] + [full(a) for a in args[3:]],
        out_specs=[eb(32), eb(48)],
        out_shape=[
            jax.ShapeDtypeStruct((E, C0), jnp.float32),
            jax.ShapeDtypeStruct((E, 3 * C1), jnp.float32),
        ],
    )(*args)


def kernel(f, pos, edge_attr, targets, edge_index, params):
    src = edge_index[0]
    dst = edge_index[1]

    rel = pos[dst] - pos[src]
    r = jnp.sqrt(jnp.sum(rel * rel, axis=-1, keepdims=True))
    dirv = rel / (r + 1e-8)
    ec = jnp.concatenate([r, edge_attr, dirv], axis=1)

    h1_in = f[:, 1:4, 0]
    h1 = (h1_in[:, :, None] * params["Win1"][0][None, None, :]).reshape(N, 48)
    h0 = jnp.zeros((N, C0), jnp.float32)

    for lp in params["layers"]:
        f0s = h0[src]
        f1s = h1[src]

        m0, m1 = _run_edge_layer(ec, f0s, f1s, lp)
        q = (h0 @ lp["Wq0"])[dst].reshape(-1, N_HEADS, HD)
        k = m0.reshape(-1, N_HEADS, HD)
        s = jnp.sum(q * k, axis=-1) / np.sqrt(HD)

        smax = jax.ops.segment_max(s, dst, num_segments=N)
        smax = jnp.where(jnp.isfinite(smax), smax, 0.0)
        ex = jnp.exp(s - smax[dst])
        denom = jax.ops.segment_sum(ex, dst, num_segments=N)
        alpha = ex / (denom[dst] + 1e-9)
        a0 = jnp.repeat(alpha, HD, axis=1)
        a1 = jnp.tile(jnp.repeat(alpha, C1 // N_HEADS, axis=1), (1, 3))
        h0 = h0 + jax.ops.segment_sum(a0 * m0, dst, num_segments=N)
        h1 = h1 + jax.ops.segment_sum(a1 * m1, dst, num_segments=N)

        n0 = jnp.abs(h0)
        h0 = jax.nn.relu(n0 @ lp["Wn0"] + lp["bn0"]) * jnp.sign(h0)
        h1v = h1.reshape(N, 3, C1)
        n1 = jnp.sqrt(jnp.sum(h1v * h1v, axis=1)) + 1e-8
        mult = jax.nn.relu(n1 @ lp["Wn1"] + lp["bn1"]) / n1
        h1 = (h1v * mult[:, None, :]).reshape(N, 48)

    fp = params["final"]
    rad_in = ec[:, :5]
    wf = jax.nn.relu(rad_in @ fp["Wr1"] + fp["br1"]) @ fp["Wr2"] + fp["br2"]
    h1cm = jnp.transpose(h1.reshape(N, 3, C1), (0, 2, 1))
    me = jnp.einsum('ecd,ec->ed', h1cm[src], wf)
    out = jax.ops.segment_sum(me, dst, num_segments=N) \
        + jnp.einsum('ncd,c->nd', h1cm, fp["self_w"])
    vec = out[None, :, :]
    loss = jnp.mean(jnp.sqrt(jnp.sum((vec - targets) ** 2, axis=-1) + 1e-5),
                    axis=-1)
    return vec, loss


# speedup vs baseline: 12.5500x; 2.9507x over previous
import functools

import jax
import jax.numpy as jnp
import numpy as np
from jax import lax
from jax.experimental import pallas as pl
from jax.experimental.pallas import tpu as pltpu
from jax.experimental.pallas import tpu_sc as plsc

N = 10000
E = 640000
NUM_LAYERS = 4
N_HEADS = 8
C0, C1 = 32, 16
RH = 32
HD = C0 // N_HEADS

BE = 2000
NW = 32
EPT = E // NW
N8 = N * N_HEADS
N8P = 81920

MESH = plsc.VectorSubcoreMesh(core_axis_name="c", subcore_axis_name="s")
SC_PARAMS = pltpu.CompilerParams(use_tc_tiling_on_sc=False,
                                 needs_layout_passes=False)
f32 = jnp.float32
i32 = jnp.int32


def _wid():
    return lax.axis_index("c") * 16 + lax.axis_index("s")


def _g_body(tabS, tabQ, srcH, dstH, outS, outQ, sbuf, dbuf, rbufS, rbufQ, sem):
    base = _wid() * EPT

    def superchunk(k, _):
        off = base + k * 2000
        pltpu.sync_copy(srcH.at[pl.ds(off, 2000)], sbuf)
        pltpu.sync_copy(dstH.at[pl.ds(off, 2000)], dbuf)
        for j in range(25):
            i0 = j * 80
            pltpu.async_copy(tabS.at[sbuf.at[pl.ds(i0, 80)]], rbufS, sem).wait()
            pltpu.sync_copy(rbufS, outS.at[pl.ds(off + i0, 80)])
            pltpu.async_copy(tabQ.at[dbuf.at[pl.ds(i0, 80)]], rbufQ, sem).wait()
            pltpu.sync_copy(rbufQ, outQ.at[pl.ds(off + i0, 80)])
        return 0

    lax.fori_loop(0, 10, superchunk, 0)


_g_call = pl.kernel(
    _g_body,
    mesh=MESH,
    compiler_params=SC_PARAMS,
    out_type=[
        jax.ShapeDtypeStruct((E, 80), f32),
        jax.ShapeDtypeStruct((E, C0), f32),
    ],
    scratch_types=[
        pltpu.VMEM((2000,), i32),
        pltpu.VMEM((2000,), i32),
        pltpu.VMEM((80, 80), f32),
        pltpu.VMEM((80, C0), f32),
        pltpu.SemaphoreType.DMA,
    ],
)


HALF = N8P // 2


def _a_body(idx8H, sflatH, outH, table, ibuf, sbuf, abuf, bbuf, spm, sem):
    cid = lax.axis_index("c")
    sid = lax.axis_index("s")
    g = sid >> 1
    q = sid & 1
    lo = q * HALF
    neg = jnp.full((16,), -1e30, f32)

    def initb(i, _):
        table[pl.ds(i * 16, 16)] = neg
        return 0

    lax.fori_loop(0, HALF // 16, initb, 0)

    iota = lax.iota(i32, 16)
    perm = (iota + 8) & 15
    base = (cid * 320000 + g * 40000) * 8

    def superchunk(k, _):
        off = base + k * 8000
        pltpu.sync_copy(idx8H.at[pl.ds(off, 8000)], ibuf)
        pltpu.sync_copy(sflatH.at[pl.ds(off, 8000)], sbuf)

        def it(i, _):
            b = i * 16
            iv = ibuf[pl.ds(b, 16)]
            sv = sbuf[pl.ds(b, 16)]
            ivs = plsc.load_gather(ibuf, [b + perm])
            svs = plsc.load_gather(sbuf, [b + perm])
            se = jnp.where(iv == ivs, jnp.maximum(sv, svs), sv)
            ivr = iv - lo
            mask = (ivr >= 0) & (ivr < HALF)
            ivc = jnp.minimum(jnp.maximum(ivr, 0), HALF - 1)
            cur = plsc.load_gather(table, [ivc])
            plsc.store_scatter(table, [ivc], jnp.maximum(cur, se), mask=mask)
            return 0

        lax.fori_loop(0, 500, it, 0)
        return 0

    lax.fori_loop(0, 40, superchunk, 0)

    pltpu.sync_copy(table, spm.at[sid])
    plsc.subcore_barrier()
    sl = g * 5120
    pltpu.sync_copy(spm.at[q, pl.ds(sl, 5120)], abuf)
    for j in range(1, 8):
        pltpu.sync_copy(spm.at[2 * j + q, pl.ds(sl, 5120)], bbuf)

        def mx(i, _):
            b = i * 16
            abuf[pl.ds(b, 16)] = jnp.maximum(abuf[pl.ds(b, 16)],
                                             bbuf[pl.ds(b, 16)])
            return 0

        lax.fori_loop(0, 320, mx, 0)
    pltpu.sync_copy(abuf, outH.at[cid, pl.ds(q * HALF + sl, 5120)])


_a_call = pl.kernel(
    _a_body,
    mesh=MESH,
    compiler_params=SC_PARAMS,
    out_type=jax.ShapeDtypeStruct((2, N8P), f32),
    scratch_types=[
        pltpu.VMEM((HALF,), f32),
        pltpu.VMEM((8000,), i32),
        pltpu.VMEM((8000,), f32),
        pltpu.VMEM((5120,), f32),
        pltpu.VMEM((5120,), f32),
        pltpu.VMEM_SHARED((16, HALF), f32),
        pltpu.SemaphoreType.DMA,
    ],
)


def _b1_body(idx8H, sflatH, pH, exH, comb, pbuf, ibuf, sbuf, ebuf, sem):
    wid = _wid()

    def ld(k, _):
        off = k * 8192
        pltpu.sync_copy(pH.at[0, pl.ds(off, 8192)], comb.at[pl.ds(off, 8192)])
        pltpu.sync_copy(pH.at[1, pl.ds(off, 8192)], pbuf)

        def mx(i, _):
            b = off + i * 16
            comb[pl.ds(b, 16)] = jnp.maximum(comb[pl.ds(b, 16)],
                                             pbuf[pl.ds(i * 16, 16)])
            return 0

        lax.fori_loop(0, 512, mx, 0)
        return 0

    lax.fori_loop(0, 10, ld, 0)

    base = wid * EPT * 8

    def superchunk(k, _):
        off = base + k * 8000
        pltpu.sync_copy(idx8H.at[pl.ds(off, 8000)], ibuf)
        pltpu.sync_copy(sflatH.at[pl.ds(off, 8000)], sbuf)

        def it(i, _):
            b = i * 16
            iv = ibuf[pl.ds(b, 16)]
            sv = sbuf[pl.ds(b, 16)]
            mv = plsc.load_gather(comb, [iv])
            ebuf[pl.ds(b, 16)] = jnp.exp(sv - mv)
            return 0

        lax.fori_loop(0, 500, it, 0)
        pltpu.sync_copy(ebuf, exH.at[pl.ds(off, 8000)])
        return 0

    lax.fori_loop(0, 20, superchunk, 0)


_b1_call = pl.kernel(
    _b1_body,
    mesh=MESH,
    compiler_params=SC_PARAMS,
    out_type=jax.ShapeDtypeStruct((E * 8,), f32),
    scratch_types=[
        pltpu.VMEM((N8P,), f32),
        pltpu.VMEM((8192,), f32),
        pltpu.VMEM((8000,), i32),
        pltpu.VMEM((8000,), f32),
        pltpu.VMEM((8000,), f32),
        pltpu.SemaphoreType.DMA,
    ],
)


def _b2_body_spmem(m0H, m1H, exH, dst2dH, outH,
                   m0b, m1b, exb, dstb, rows, acc, sem):
    cid = lax.axis_index("c")
    sid = lax.axis_index("s")
    wid = cid * 16 + sid
    iota = lax.iota(i32, 16)
    io4 = iota >> 2
    io2 = iota >> 1
    io8 = iota & 7
    lt8 = iota < 8
    zeros = jnp.zeros((16,), f32)

    def zrow(i, _):
        r = i // 6
        c = (i % 6) * 16
        rows[r, pl.ds(c, 16)] = zeros
        return 0

    lax.fori_loop(0, 125 * 6, zrow, 0)
    for t in range(5):
        pltpu.sync_copy(rows, acc.at[pl.ds(sid * 625 + t * 125, 125)])
    plsc.subcore_barrier()

    base_e = wid * EPT

    def superchunk(k, _):
        e0 = base_e + k * 500
        pltpu.sync_copy(m0H.at[pl.ds(e0 * 32, 16000)], m0b)
        pltpu.sync_copy(m1H.at[pl.ds(e0 * 48, 24000)], m1b)
        pltpu.sync_copy(exH.at[pl.ds(e0 * 8, 4000)], exb)
        pltpu.sync_copy(dst2dH.at[pl.ds(e0 // 125, 4)], dstb)
        for sub in range(4):
            def edge_it(j, _):
                e = sub * 125 + j
                e8 = e * 8
                ex4a = plsc.load_gather(exb, [e8 + io4])
                ex4b = plsc.load_gather(exb, [e8 + 4 + io4])
                ex2 = plsc.load_gather(exb, [e8 + io2])
                ext = jnp.where(lt8, plsc.load_gather(exb, [e8 + io8]), 0.0)
                m32 = e * 32
                m48 = e * 48
                rows[j, pl.ds(0, 16)] = m0b[pl.ds(m32, 16)] * ex4a
                rows[j, pl.ds(16, 16)] = m0b[pl.ds(m32 + 16, 16)] * ex4b
                rows[j, pl.ds(32, 16)] = m1b[pl.ds(m48, 16)] * ex2
                rows[j, pl.ds(48, 16)] = m1b[pl.ds(m48 + 16, 16)] * ex2
                rows[j, pl.ds(64, 16)] = m1b[pl.ds(m48 + 32, 16)] * ex2
                rows[j, pl.ds(80, 16)] = ext
                return 0

            lax.fori_loop(0, 125, edge_it, 0)
            pltpu.sync_copy(rows, acc.at[dstb.at[sub]], add=True)
        return 0

    lax.fori_loop(0, 40, superchunk, 0)
    plsc.subcore_barrier()
    pltpu.sync_copy(acc.at[pl.ds(sid * 625, 625)],
                    outH.at[cid, pl.ds(sid * 625, 625)])


_b2_call = pl.kernel(
    _b2_body_spmem,
    mesh=MESH,
    compiler_params=SC_PARAMS,
    out_type=jax.ShapeDtypeStruct((2, N, 96), f32),
    scratch_types=[
        pltpu.VMEM((16000,), f32),
        pltpu.VMEM((24000,), f32),
        pltpu.VMEM((4000,), f32),
        pltpu.VMEM((4, 125), i32),
        pltpu.VMEM((125, 96), f32),
        pltpu.VMEM_SHARED((N, 96), f32),
        pltpu.SemaphoreType.DMA,
    ],
)


def _edge_kernel(os_ref, oq_ref, ec_ref,
                 wr1_ref, br1_ref, wr2_ref, br2_ref,
                 w00_ref, w10_ref, w01_ref, w11_ref, summ_ref,
                 m0_ref, m1_ref, s_ref):
    osv = os_ref[...]
    f0 = osv[:, :32]
    f1 = osv[:, 32:80]
    q = oq_ref[...]
    ec = ec_ref[...]

    dot = lambda a, b: jnp.dot(a, b, preferred_element_type=f32)

    rad = ec
    hwr = jnp.maximum(dot(rad, wr1_ref[...]) + br1_ref[...], 0.0)
    w = dot(hwr, wr2_ref[...]) + br2_ref[...]
    w0 = w[:, :C0]
    w1 = w[:, C0:C0 + C1]

    d0 = ec[:, 5:6]
    d1 = ec[:, 6:7]
    d2 = ec[:, 7:8]
    f1a = f1[:, 0:16]
    f1b = f1[:, 16:32]
    f1c = f1[:, 32:48]
    bf = lambda x: x.astype(jnp.bfloat16).astype(f32)
    f1d = bf(f1a) * bf(d0) + bf(f1b) * bf(d1) + bf(f1c) * bf(d2)

    m0 = (dot(f0, w00_ref[...]) + dot(f1d, w10_ref[...])) * w0
    g = dot(f0, w01_ref[...])
    w11 = w11_ref[...]
    m1a = (dot(f1a, w11) + g * d0) * w1
    m1b = (dot(f1b, w11) + g * d1) * w1
    m1c = (dot(f1c, w11) + g * d2) * w1

    qk = q * m0
    s = jnp.dot(qk, summ_ref[...], preferred_element_type=f32,
                precision=jax.lax.Precision.HIGHEST) \
        * np.float32(1.0 / np.sqrt(HD))

    m0_ref[...] = m0
    m1_ref[...] = jnp.concatenate([m1a, m1b, m1c], axis=1)
    s_ref[...] = s


def _run_edge_layer(osv, oqv, ec, lp, summ):
    wr1 = jnp.zeros((8, RH), f32).at[:5, :].set(lp["Wr1"])
    wr2 = lp["Wr2"][:, :C0 + C1]
    br2 = lp["br2"][:C0 + C1]

    grid = (E // BE,)
    eb = lambda w: pl.BlockSpec((BE, w), lambda i: (i, 0))
    full = lambda a: pl.BlockSpec(a.shape, lambda i: (0,) * a.ndim)
    args = (osv, oqv, ec, wr1, lp["br1"], wr2, br2, lp["W00"], lp["W10"],
            lp["W01"], lp["W11"], summ)
    return pl.pallas_call(
        _edge_kernel,
        grid=grid,
        in_specs=[eb(80), eb(32), eb(8)] + [full(a) for a in args[3:]],
        out_specs=[eb(32), eb(48), eb(8)],
        out_shape=[
            jax.ShapeDtypeStruct((E, C0), f32),
            jax.ShapeDtypeStruct((E, 3 * C1), f32),
            jax.ShapeDtypeStruct((E, N_HEADS), f32),
        ],
    )(*args)


def kernel(f, pos, edge_attr, targets, edge_index, params):
    src = edge_index[0]
    dst = edge_index[1]

    rel = pos[dst] - pos[src]
    r = jnp.sqrt(jnp.sum(rel * rel, axis=-1, keepdims=True))
    dirv = rel / (r + 1e-8)
    ec = jnp.concatenate([r, edge_attr, dirv], axis=1)

    idx8 = (dst[:, None] * 8 + jnp.arange(8, dtype=i32)[None, :]).reshape(-1)
    dst2d = dst.reshape(E // 125, 125)
    summ = jnp.repeat(jnp.eye(N_HEADS, dtype=f32), HD, axis=0)

    h1_in = f[:, 1:4, 0]
    h1 = (h1_in[:, :, None] * params["Win1"][0][None, None, :]).reshape(N, 48)
    h0 = jnp.zeros((N, C0), f32)

    for lp in params["layers"]:
        tabS = jnp.concatenate([h0, h1], axis=1)
        tabQ = h0 @ lp["Wq0"]

        osv, oqv = _g_call(tabS, tabQ, src, dst)
        m0, m1, s = _run_edge_layer(osv, oqv, ec, lp, summ)

        sflat = s.reshape(-1)
        part = _a_call(idx8, sflat)
        ex = _b1_call(idx8, sflat, part)
        acc = _b2_call(m0.reshape(-1), m1.reshape(-1), ex, dst2d)

        accsum = acc[0] + acc[1]
        sum0 = accsum[:, :32]
        sum1 = accsum[:, 32:80]
        den = accsum[:, 80:88]
        den0 = jnp.repeat(den, HD, axis=1) + 1e-9
        den1 = jnp.tile(jnp.repeat(den, C1 // N_HEADS, axis=1), (1, 3)) + 1e-9
        h0 = h0 + sum0 / den0
        h1 = h1 + sum1 / den1

        n0 = jnp.abs(h0)
        h0 = jax.nn.relu(n0 @ lp["Wn0"] + lp["bn0"]) * jnp.sign(h0)
        h1v = h1.reshape(N, 3, C1)
        n1 = jnp.sqrt(jnp.sum(h1v * h1v, axis=1)) + 1e-8
        mult = jax.nn.relu(n1 @ lp["Wn1"] + lp["bn1"]) / n1
        h1 = (h1v * mult[:, None, :]).reshape(N, 48)

    fp = params["final"]
    rad_in = ec[:, :5]
    wf = jax.nn.relu(rad_in @ fp["Wr1"] + fp["br1"]) @ fp["Wr2"] + fp["br2"]
    h1cm = jnp.transpose(h1.reshape(N, 3, C1), (0, 2, 1))
    me = jnp.einsum('ecd,ec->ed', h1cm[src], wf)
    out = jax.ops.segment_sum(me, dst, num_segments=N) \
        + jnp.einsum('ncd,c->nd', h1cm, fp["self_w"])
    vec = out[None, :, :]
    loss = jnp.mean(jnp.sqrt(jnp.sum((vec - targets) ** 2, axis=-1) + 1e-5),
                    axis=-1)
    return vec, loss


# speedup vs baseline: 13.0043x; 1.0362x over previous
import functools

import jax
import jax.numpy as jnp
import numpy as np
from jax import lax
from jax.experimental import pallas as pl
from jax.experimental.pallas import tpu as pltpu
from jax.experimental.pallas import tpu_sc as plsc

N = 10000
E = 640000
NUM_LAYERS = 4
N_HEADS = 8
C0, C1 = 32, 16
RH = 32
HD = C0 // N_HEADS

BE = 2000
NW = 32
EPT = E // NW
N8 = N * N_HEADS
N8P = 81920

MESH = plsc.VectorSubcoreMesh(core_axis_name="c", subcore_axis_name="s")
SC_PARAMS = pltpu.CompilerParams(use_tc_tiling_on_sc=False,
                                 needs_layout_passes=False)
f32 = jnp.float32
i32 = jnp.int32


def _wid():
    return lax.axis_index("c") * 16 + lax.axis_index("s")


def _g_body(tabS, tabQ, srcH, dstH, outS, outQ,
            sbuf, dbuf, rS0, rQ0, rS1, rQ1, semS, semQ):
    base = _wid() * EPT
    rS = (rS0, rS1)
    rQ = (rQ0, rQ1)

    def superchunk(k, _):
        off = base + k * 2000
        pltpu.sync_copy(srcH.at[pl.ds(off, 2000)], sbuf)
        pltpu.sync_copy(dstH.at[pl.ds(off, 2000)], dbuf)
        cs = pltpu.async_copy(tabS.at[sbuf.at[pl.ds(0, 80)]], rS[0], semS)
        cq = pltpu.async_copy(tabQ.at[dbuf.at[pl.ds(0, 80)]], rQ[0], semQ)
        for j in range(25):
            p = j & 1
            np_ = (j + 1) & 1
            if j < 24:
                i1 = (j + 1) * 80
                ns = pltpu.async_copy(tabS.at[sbuf.at[pl.ds(i1, 80)]],
                                      rS[np_], semS)
                nq = pltpu.async_copy(tabQ.at[dbuf.at[pl.ds(i1, 80)]],
                                      rQ[np_], semQ)
            cs.wait()
            cq.wait()
            i0 = j * 80
            pltpu.sync_copy(rS[p], outS.at[pl.ds(off + i0, 80)])
            pltpu.sync_copy(rQ[p], outQ.at[pl.ds(off + i0, 80)])
            if j < 24:
                cs, cq = ns, nq
        return 0

    lax.fori_loop(0, 10, superchunk, 0)


_g_call = pl.kernel(
    _g_body,
    mesh=MESH,
    compiler_params=SC_PARAMS,
    out_type=[
        jax.ShapeDtypeStruct((E, 80), f32),
        jax.ShapeDtypeStruct((E, C0), f32),
    ],
    scratch_types=[
        pltpu.VMEM((2000,), i32),
        pltpu.VMEM((2000,), i32),
        pltpu.VMEM((80, 80), f32),
        pltpu.VMEM((80, C0), f32),
        pltpu.VMEM((80, 80), f32),
        pltpu.VMEM((80, C0), f32),
        pltpu.SemaphoreType.DMA,
        pltpu.SemaphoreType.DMA,
    ],
)


HALF = N8P // 2


def _a_body(idx8H, sflatH, outH, table, ibuf, sbuf, abuf, bbuf, spm, sem):
    cid = lax.axis_index("c")
    sid = lax.axis_index("s")
    g = sid >> 1
    q = sid & 1
    lo = q * HALF
    neg = jnp.full((16,), -1e30, f32)

    def initb(i, _):
        table[pl.ds(i * 16, 16)] = neg
        return 0

    lax.fori_loop(0, HALF // 16, initb, 0)

    iota = lax.iota(i32, 16)
    perm = (iota + 8) & 15
    base = (cid * 320000 + g * 40000) * 8

    def superchunk(k, _):
        off = base + k * 8000
        pltpu.sync_copy(idx8H.at[pl.ds(off, 8000)], ibuf)
        pltpu.sync_copy(sflatH.at[pl.ds(off, 8000)], sbuf)

        def it(i, _):
            b = i * 16
            iv = ibuf[pl.ds(b, 16)]
            sv = sbuf[pl.ds(b, 16)]
            ivs = plsc.load_gather(ibuf, [b + perm])
            svs = plsc.load_gather(sbuf, [b + perm])
            se = jnp.where(iv == ivs, jnp.maximum(sv, svs), sv)
            ivr = iv - lo
            mask = (ivr >= 0) & (ivr < HALF)
            ivc = jnp.minimum(jnp.maximum(ivr, 0), HALF - 1)
            cur = plsc.load_gather(table, [ivc])
            plsc.store_scatter(table, [ivc], jnp.maximum(cur, se), mask=mask)
            return 0

        lax.fori_loop(0, 500, it, 0)
        return 0

    lax.fori_loop(0, 40, superchunk, 0)

    pltpu.sync_copy(table, spm.at[sid])
    plsc.subcore_barrier()
    sl = g * 5120
    pltpu.sync_copy(spm.at[q, pl.ds(sl, 5120)], abuf)
    for j in range(1, 8):
        pltpu.sync_copy(spm.at[2 * j + q, pl.ds(sl, 5120)], bbuf)

        def mx(i, _):
            b = i * 16
            abuf[pl.ds(b, 16)] = jnp.maximum(abuf[pl.ds(b, 16)],
                                             bbuf[pl.ds(b, 16)])
            return 0

        lax.fori_loop(0, 320, mx, 0)
    pltpu.sync_copy(abuf, outH.at[cid, pl.ds(q * HALF + sl, 5120)])


_a_call = pl.kernel(
    _a_body,
    mesh=MESH,
    compiler_params=SC_PARAMS,
    out_type=jax.ShapeDtypeStruct((2, N8P), f32),
    scratch_types=[
        pltpu.VMEM((HALF,), f32),
        pltpu.VMEM((8000,), i32),
        pltpu.VMEM((8000,), f32),
        pltpu.VMEM((5120,), f32),
        pltpu.VMEM((5120,), f32),
        pltpu.VMEM_SHARED((16, HALF), f32),
        pltpu.SemaphoreType.DMA,
    ],
)


def _b1_body(idx8H, sflatH, pH, exH, comb, pbuf, ibuf, sbuf, ebuf, sem):
    wid = _wid()

    def ld(k, _):
        off = k * 8192
        pltpu.sync_copy(pH.at[0, pl.ds(off, 8192)], comb.at[pl.ds(off, 8192)])
        pltpu.sync_copy(pH.at[1, pl.ds(off, 8192)], pbuf)

        def mx(i, _):
            b = off + i * 16
            comb[pl.ds(b, 16)] = jnp.maximum(comb[pl.ds(b, 16)],
                                             pbuf[pl.ds(i * 16, 16)])
            return 0

        lax.fori_loop(0, 512, mx, 0)
        return 0

    lax.fori_loop(0, 10, ld, 0)

    base = wid * EPT * 8

    def superchunk(k, _):
        off = base + k * 8000
        pltpu.sync_copy(idx8H.at[pl.ds(off, 8000)], ibuf)
        pltpu.sync_copy(sflatH.at[pl.ds(off, 8000)], sbuf)

        def it(i, _):
            b = i * 16
            iv = ibuf[pl.ds(b, 16)]
            sv = sbuf[pl.ds(b, 16)]
            mv = plsc.load_gather(comb, [iv])
            ebuf[pl.ds(b, 16)] = jnp.exp(sv - mv)
            return 0

        lax.fori_loop(0, 500, it, 0)
        pltpu.sync_copy(ebuf, exH.at[pl.ds(off, 8000)])
        return 0

    lax.fori_loop(0, 20, superchunk, 0)


_b1_call = pl.kernel(
    _b1_body,
    mesh=MESH,
    compiler_params=SC_PARAMS,
    out_type=jax.ShapeDtypeStruct((E * 8,), f32),
    scratch_types=[
        pltpu.VMEM((N8P,), f32),
        pltpu.VMEM((8192,), f32),
        pltpu.VMEM((8000,), i32),
        pltpu.VMEM((8000,), f32),
        pltpu.VMEM((8000,), f32),
        pltpu.SemaphoreType.DMA,
    ],
)


def _b2_body(exH, m0H, m1H, dst2dH, outH,
             exb, m0b, m1b, dstb, rows, acc, sem):
    cid = lax.axis_index("c")
    sid = lax.axis_index("s")
    wid = cid * 16 + sid
    iota = lax.iota(i32, 16)
    io4 = iota >> 2
    io2 = iota >> 1
    io8 = iota & 7
    lt8 = iota < 8
    zeros = jnp.zeros((16,), f32)

    def zrow(i, _):
        r = i // 6
        c = (i % 6) * 16
        rows[r, pl.ds(c, 16)] = zeros
        return 0

    lax.fori_loop(0, 125 * 6, zrow, 0)
    for t in range(5):
        pltpu.sync_copy(rows, acc.at[pl.ds(sid * 625 + t * 125, 125)])
    plsc.subcore_barrier()

    base_e = wid * EPT

    def superchunk(k, _):
        e0 = base_e + k * 500
        pltpu.sync_copy(exH.at[pl.ds(e0 * 8, 4000)], exb)
        pltpu.sync_copy(m0H.at[pl.ds(e0 * 32, 16000)], m0b)
        pltpu.sync_copy(m1H.at[pl.ds(e0 * 48, 24000)], m1b)
        pltpu.sync_copy(dst2dH.at[pl.ds(e0 // 125, 4)], dstb)
        for sub in range(4):
            def edge_it(j, _):
                e = sub * 125 + j
                e8 = e * 8
                ex4a = plsc.load_gather(exb, [e8 + io4])
                ex4b = plsc.load_gather(exb, [e8 + 4 + io4])
                ex2 = plsc.load_gather(exb, [e8 + io2])
                ext = jnp.where(lt8, plsc.load_gather(exb, [e8 + io8]), 0.0)
                m32 = e * 32
                m48 = e * 48
                rows[j, pl.ds(0, 16)] = m0b[pl.ds(m32, 16)] * ex4a
                rows[j, pl.ds(16, 16)] = m0b[pl.ds(m32 + 16, 16)] * ex4b
                rows[j, pl.ds(32, 16)] = m1b[pl.ds(m48, 16)] * ex2
                rows[j, pl.ds(48, 16)] = m1b[pl.ds(m48 + 16, 16)] * ex2
                rows[j, pl.ds(64, 16)] = m1b[pl.ds(m48 + 32, 16)] * ex2
                rows[j, pl.ds(80, 16)] = ext
                return 0

            lax.fori_loop(0, 125, edge_it, 0)
            pltpu.sync_copy(rows, acc.at[dstb.at[sub]], add=True)
        return 0

    lax.fori_loop(0, 40, superchunk, 0)
    plsc.subcore_barrier()
    pltpu.sync_copy(acc.at[pl.ds(sid * 625, 625)],
                    outH.at[cid, pl.ds(sid * 625, 625)])


_b2_call = pl.kernel(
    _b2_body,
    mesh=MESH,
    compiler_params=SC_PARAMS,
    out_type=jax.ShapeDtypeStruct((2, N, 96), f32),
    scratch_types=[
        pltpu.VMEM((4000,), f32),
        pltpu.VMEM((16000,), f32),
        pltpu.VMEM((24000,), f32),
        pltpu.VMEM((4, 125), i32),
        pltpu.VMEM((125, 96), f32),
        pltpu.VMEM_SHARED((N, 96), f32),
        pltpu.SemaphoreType.DMA,
    ],
)


def _edge_kernel(os_ref, oq_ref, ec_ref,
                 wr1_ref, br1_ref, wr2_ref, br2_ref,
                 w00_ref, w10_ref, w01_ref, w11_ref, summ_ref,
                 m0_ref, m1_ref, s_ref):
    osv = os_ref[...]
    f0 = osv[:, :32]
    f1 = osv[:, 32:80]
    q = oq_ref[...]
    ec = ec_ref[...]

    dot = lambda a, b: jnp.dot(a, b, preferred_element_type=f32)

    rad = ec
    hwr = jnp.maximum(dot(rad, wr1_ref[...]) + br1_ref[...], 0.0)
    w = dot(hwr, wr2_ref[...]) + br2_ref[...]
    w0 = w[:, :C0]
    w1 = w[:, C0:C0 + C1]

    d0 = ec[:, 5:6]
    d1 = ec[:, 6:7]
    d2 = ec[:, 7:8]
    f1a = f1[:, 0:16]
    f1b = f1[:, 16:32]
    f1c = f1[:, 32:48]
    bf = lambda x: x.astype(jnp.bfloat16).astype(f32)
    f1d = bf(f1a) * bf(d0) + bf(f1b) * bf(d1) + bf(f1c) * bf(d2)

    m0 = (dot(f0, w00_ref[...]) + dot(f1d, w10_ref[...])) * w0
    g = dot(f0, w01_ref[...])
    w11 = w11_ref[...]
    m1a = (dot(f1a, w11) + g * d0) * w1
    m1b = (dot(f1b, w11) + g * d1) * w1
    m1c = (dot(f1c, w11) + g * d2) * w1

    qk = q * m0
    s = jnp.dot(qk, summ_ref[...], preferred_element_type=f32,
                precision=jax.lax.Precision.HIGHEST) \
        * np.float32(1.0 / np.sqrt(HD))

    m0_ref[...] = m0
    m1_ref[...] = jnp.concatenate([m1a, m1b, m1c], axis=1)
    s_ref[...] = s


def _run_edge_layer(osv, oqv, ec, lp, summ):
    wr1 = jnp.zeros((8, RH), f32).at[:5, :].set(lp["Wr1"])
    wr2 = lp["Wr2"][:, :C0 + C1]
    br2 = lp["br2"][:C0 + C1]

    grid = (E // BE,)
    eb = lambda w: pl.BlockSpec((BE, w), lambda i: (i, 0))
    full = lambda a: pl.BlockSpec(a.shape, lambda i: (0,) * a.ndim)
    args = (osv, oqv, ec, wr1, lp["br1"], wr2, br2, lp["W00"], lp["W10"],
            lp["W01"], lp["W11"], summ)
    return pl.pallas_call(
        _edge_kernel,
        grid=grid,
        in_specs=[eb(80), eb(32), eb(8)] + [full(a) for a in args[3:]],
        out_specs=[eb(32), eb(48), eb(8)],
        out_shape=[
            jax.ShapeDtypeStruct((E, C0), f32),
            jax.ShapeDtypeStruct((E, 3 * C1), f32),
            jax.ShapeDtypeStruct((E, N_HEADS), f32),
        ],
    )(*args)


def kernel(f, pos, edge_attr, targets, edge_index, params):
    src = edge_index[0]
    dst = edge_index[1]

    rel = pos[dst] - pos[src]
    r = jnp.sqrt(jnp.sum(rel * rel, axis=-1, keepdims=True))
    dirv = rel / (r + 1e-8)
    ec = jnp.concatenate([r, edge_attr, dirv], axis=1)

    idx8 = (dst[:, None] * 8 + jnp.arange(8, dtype=i32)[None, :]).reshape(-1)
    dst2d = dst.reshape(E // 125, 125)
    summ = jnp.repeat(jnp.eye(N_HEADS, dtype=f32), HD, axis=0)

    h1_in = f[:, 1:4, 0]
    h1 = (h1_in[:, :, None] * params["Win1"][0][None, None, :]).reshape(N, 48)
    h0 = jnp.zeros((N, C0), f32)

    for lp in params["layers"]:
        tabS = jnp.concatenate([h0, h1], axis=1)
        tabQ = h0 @ lp["Wq0"]

        osv, oqv = _g_call(tabS, tabQ, src, dst)
        m0, m1, s = _run_edge_layer(osv, oqv, ec, lp, summ)

        sflat = s.reshape(-1)
        part = _a_call(idx8, sflat)
        ex = _b1_call(idx8, sflat, part)
        acc = _b2_call(ex, m0.reshape(-1), m1.reshape(-1), dst2d)

        accsum = acc[0] + acc[1]
        sum0 = accsum[:, :32]
        sum1 = accsum[:, 32:80]
        den = accsum[:, 80:88]
        den0 = jnp.repeat(den, HD, axis=1) + 1e-9
        den1 = jnp.tile(jnp.repeat(den, C1 // N_HEADS, axis=1), (1, 3)) + 1e-9
        h0 = h0 + sum0 / den0
        h1 = h1 + sum1 / den1

        n0 = jnp.abs(h0)
        h0 = jax.nn.relu(n0 @ lp["Wn0"] + lp["bn0"]) * jnp.sign(h0)
        h1v = h1.reshape(N, 3, C1)
        n1 = jnp.sqrt(jnp.sum(h1v * h1v, axis=1)) + 1e-8
        mult = jax.nn.relu(n1 @ lp["Wn1"] + lp["bn1"]) / n1
        h1 = (h1v * mult[:, None, :]).reshape(N, 48)

    fp = params["final"]
    rad_in = ec[:, :5]
    wf = jax.nn.relu(rad_in @ fp["Wr1"] + fp["br1"]) @ fp["Wr2"] + fp["br2"]
    h1cm = jnp.transpose(h1.reshape(N, 3, C1), (0, 2, 1))
    me = jnp.einsum('ecd,ec->ed', h1cm[src], wf)
    out = jax.ops.segment_sum(me, dst, num_segments=N) \
        + jnp.einsum('ncd,c->nd', h1cm, fp["self_w"])
    vec = out[None, :, :]
    loss = jnp.mean(jnp.sqrt(jnp.sum((vec - targets) ** 2, axis=-1) + 1e-5),
                    axis=-1)
    return vec, loss


# speedup vs baseline: 16.7808x; 1.2904x over previous
import functools

import jax
import jax.numpy as jnp
import numpy as np
from jax import lax
from jax.experimental import pallas as pl
from jax.experimental.pallas import tpu as pltpu
from jax.experimental.pallas import tpu_sc as plsc

N = 10000
E = 640000
NUM_LAYERS = 4
N_HEADS = 8
C0, C1 = 32, 16
RH = 32
HD = C0 // N_HEADS

BE = 2000
NW = 32
EPT = E // NW
N8 = N * N_HEADS
N8P = 81920

MESH = plsc.VectorSubcoreMesh(core_axis_name="c", subcore_axis_name="s")
SC_PARAMS = pltpu.CompilerParams(use_tc_tiling_on_sc=False,
                                 needs_layout_passes=False)
f32 = jnp.float32
i32 = jnp.int32


def _wid():
    return lax.axis_index("c") * 16 + lax.axis_index("s")


def _g_body_w(tabS, tabQ, srcH, dstH, outS, outQ,
              sbuf, dbuf, rS0, rQ0, rS1, rQ1, semS, semQ):
    base = _wid() * EPT
    rS = (rS0, rS1)
    rQ = (rQ0, rQ1)

    def superchunk(k, _):
        off = base + k * 2000
        pltpu.sync_copy(srcH.at[pl.ds(off, 2000)], sbuf)
        pltpu.sync_copy(dstH.at[pl.ds(off, 2000)], dbuf)
        cs = pltpu.async_copy(tabS.at[sbuf.at[pl.ds(0, 80)]], rS[0], semS)
        cq = pltpu.async_copy(tabQ.at[dbuf.at[pl.ds(0, 80)]], rQ[0], semQ)
        for j in range(25):
            p = j & 1
            np_ = (j + 1) & 1
            if j < 24:
                i1 = (j + 1) * 80
                ns = pltpu.async_copy(tabS.at[sbuf.at[pl.ds(i1, 80)]],
                                      rS[np_], semS)
                nq = pltpu.async_copy(tabQ.at[dbuf.at[pl.ds(i1, 80)]],
                                      rQ[np_], semQ)
            cs.wait()
            cq.wait()
            i0 = j * 80
            pltpu.sync_copy(rS[p], outS.at[pl.ds(off + i0, 80)])
            pltpu.sync_copy(rQ[p], outQ.at[pl.ds(off + i0, 80)])
            if j < 24:
                cs, cq = ns, nq
        return 0

    lax.fori_loop(0, 10, superchunk, 0)


def _make_g(ds_, dq_):
    return pl.kernel(
        _g_body_w,
        mesh=MESH,
        compiler_params=SC_PARAMS,
        out_type=[
            jax.ShapeDtypeStruct((E, ds_), f32),
            jax.ShapeDtypeStruct((E, dq_), f32),
        ],
        scratch_types=[
            pltpu.VMEM((2000,), i32),
            pltpu.VMEM((2000,), i32),
            pltpu.VMEM((80, ds_), f32),
            pltpu.VMEM((80, dq_), f32),
            pltpu.VMEM((80, ds_), f32),
            pltpu.VMEM((80, dq_), f32),
            pltpu.SemaphoreType.DMA,
            pltpu.SemaphoreType.DMA,
        ],
    )


_g_call = _make_g(80, C0)
_g_pos_call = _make_g(8, 8)
_g_fin_call = _make_g(48, 8)


HALF = N8P // 2


def _a_body(idx8H, sflatH, outH, table, ibuf, sbuf, abuf, bbuf, spm, sem):
    cid = lax.axis_index("c")
    sid = lax.axis_index("s")
    g = sid >> 1
    q = sid & 1
    lo = q * HALF
    neg = jnp.full((16,), -1e30, f32)

    def initb(i, _):
        table[pl.ds(i * 16, 16)] = neg
        return 0

    lax.fori_loop(0, HALF // 16, initb, 0)

    iota = lax.iota(i32, 16)
    perm = (iota + 8) & 15
    base = (cid * 320000 + g * 40000) * 8

    def superchunk(k, _):
        off = base + k * 8000
        pltpu.sync_copy(idx8H.at[pl.ds(off, 8000)], ibuf)
        pltpu.sync_copy(sflatH.at[pl.ds(off, 8000)], sbuf)

        def it(i, _):
            b = i * 16
            iv = ibuf[pl.ds(b, 16)]
            sv = sbuf[pl.ds(b, 16)]
            ivs = plsc.load_gather(ibuf, [b + perm])
            svs = plsc.load_gather(sbuf, [b + perm])
            se = jnp.where(iv == ivs, jnp.maximum(sv, svs), sv)
            ivr = iv - lo
            mask = (ivr >= 0) & (ivr < HALF)
            ivc = jnp.minimum(jnp.maximum(ivr, 0), HALF - 1)
            cur = plsc.load_gather(table, [ivc])
            plsc.store_scatter(table, [ivc], jnp.maximum(cur, se), mask=mask)
            return 0

        lax.fori_loop(0, 500, it, 0)
        return 0

    lax.fori_loop(0, 40, superchunk, 0)

    pltpu.sync_copy(table, spm.at[sid])
    plsc.subcore_barrier()
    sl = g * 5120
    pltpu.sync_copy(spm.at[q, pl.ds(sl, 5120)], abuf)
    for j in range(1, 8):
        pltpu.sync_copy(spm.at[2 * j + q, pl.ds(sl, 5120)], bbuf)

        def mx(i, _):
            b = i * 16
            abuf[pl.ds(b, 16)] = jnp.maximum(abuf[pl.ds(b, 16)],
                                             bbuf[pl.ds(b, 16)])
            return 0

        lax.fori_loop(0, 320, mx, 0)
    pltpu.sync_copy(abuf, outH.at[cid, pl.ds(q * HALF + sl, 5120)])


_a_call = pl.kernel(
    _a_body,
    mesh=MESH,
    compiler_params=SC_PARAMS,
    out_type=jax.ShapeDtypeStruct((2, N8P), f32),
    scratch_types=[
        pltpu.VMEM((HALF,), f32),
        pltpu.VMEM((8000,), i32),
        pltpu.VMEM((8000,), f32),
        pltpu.VMEM((5120,), f32),
        pltpu.VMEM((5120,), f32),
        pltpu.VMEM_SHARED((16, HALF), f32),
        pltpu.SemaphoreType.DMA,
    ],
)


def _b1_body(idx8H, sflatH, pH, exH, comb, pbuf, ibuf, sbuf, ebuf, sem):
    wid = _wid()

    def ld(k, _):
        off = k * 8192
        pltpu.sync_copy(pH.at[0, pl.ds(off, 8192)], comb.at[pl.ds(off, 8192)])
        pltpu.sync_copy(pH.at[1, pl.ds(off, 8192)], pbuf)

        def mx(i, _):
            b = off + i * 16
            comb[pl.ds(b, 16)] = jnp.maximum(comb[pl.ds(b, 16)],
                                             pbuf[pl.ds(i * 16, 16)])
            return 0

        lax.fori_loop(0, 512, mx, 0)
        return 0

    lax.fori_loop(0, 10, ld, 0)

    base = wid * EPT * 8

    def superchunk(k, _):
        off = base + k * 8000
        pltpu.sync_copy(idx8H.at[pl.ds(off, 8000)], ibuf)
        pltpu.sync_copy(sflatH.at[pl.ds(off, 8000)], sbuf)

        def it(i, _):
            b = i * 16
            iv = ibuf[pl.ds(b, 16)]
            sv = sbuf[pl.ds(b, 16)]
            mv = plsc.load_gather(comb, [iv])
            ebuf[pl.ds(b, 16)] = jnp.exp(sv - mv)
            return 0

        lax.fori_loop(0, 500, it, 0)
        pltpu.sync_copy(ebuf, exH.at[pl.ds(off, 8000)])
        return 0

    lax.fori_loop(0, 20, superchunk, 0)


_b1_call = pl.kernel(
    _b1_body,
    mesh=MESH,
    compiler_params=SC_PARAMS,
    out_type=jax.ShapeDtypeStruct((E * 8,), f32),
    scratch_types=[
        pltpu.VMEM((N8P,), f32),
        pltpu.VMEM((8192,), f32),
        pltpu.VMEM((8000,), i32),
        pltpu.VMEM((8000,), f32),
        pltpu.VMEM((8000,), f32),
        pltpu.SemaphoreType.DMA,
    ],
)


def _b2_body(exH, m0H, m1H, dst2dH, outH,
             exb, m0b, m1b, dstb, rows, acc, sem):
    cid = lax.axis_index("c")
    sid = lax.axis_index("s")
    wid = cid * 16 + sid
    iota = lax.iota(i32, 16)
    io4 = iota >> 2
    io2 = iota >> 1
    io8 = iota & 7
    lt8 = iota < 8
    zeros = jnp.zeros((16,), f32)

    def zrow(i, _):
        r = i // 6
        c = (i % 6) * 16
        rows[r, pl.ds(c, 16)] = zeros
        return 0

    lax.fori_loop(0, 125 * 6, zrow, 0)
    for t in range(5):
        pltpu.sync_copy(rows, acc.at[pl.ds(sid * 625 + t * 125, 125)])
    plsc.subcore_barrier()

    base_e = wid * EPT

    def superchunk(k, _):
        e0 = base_e + k * 500
        pltpu.sync_copy(exH.at[pl.ds(e0 * 8, 4000)], exb)
        pltpu.sync_copy(m0H.at[pl.ds(e0 * 32, 16000)], m0b)
        pltpu.sync_copy(m1H.at[pl.ds(e0 * 48, 24000)], m1b)
        pltpu.sync_copy(dst2dH.at[pl.ds(e0 // 125, 4)], dstb)
        for sub in range(4):
            def edge_it(j, _):
                e = sub * 125 + j
                e8 = e * 8
                ex4a = plsc.load_gather(exb, [e8 + io4])
                ex4b = plsc.load_gather(exb, [e8 + 4 + io4])
                ex2 = plsc.load_gather(exb, [e8 + io2])
                ext = jnp.where(lt8, plsc.load_gather(exb, [e8 + io8]), 0.0)
                m32 = e * 32
                m48 = e * 48
                rows[j, pl.ds(0, 16)] = m0b[pl.ds(m32, 16)] * ex4a
                rows[j, pl.ds(16, 16)] = m0b[pl.ds(m32 + 16, 16)] * ex4b
                rows[j, pl.ds(32, 16)] = m1b[pl.ds(m48, 16)] * ex2
                rows[j, pl.ds(48, 16)] = m1b[pl.ds(m48 + 16, 16)] * ex2
                rows[j, pl.ds(64, 16)] = m1b[pl.ds(m48 + 32, 16)] * ex2
                rows[j, pl.ds(80, 16)] = ext
                return 0

            lax.fori_loop(0, 125, edge_it, 0)
            pltpu.sync_copy(rows, acc.at[dstb.at[sub]], add=True)
        return 0

    lax.fori_loop(0, 40, superchunk, 0)
    plsc.subcore_barrier()
    pltpu.sync_copy(acc.at[pl.ds(sid * 625, 625)],
                    outH.at[cid, pl.ds(sid * 625, 625)])


_b2_call = pl.kernel(
    _b2_body,
    mesh=MESH,
    compiler_params=SC_PARAMS,
    out_type=jax.ShapeDtypeStruct((2, N, 96), f32),
    scratch_types=[
        pltpu.VMEM((4000,), f32),
        pltpu.VMEM((16000,), f32),
        pltpu.VMEM((24000,), f32),
        pltpu.VMEM((4, 125), i32),
        pltpu.VMEM((125, 96), f32),
        pltpu.VMEM_SHARED((N, 96), f32),
        pltpu.SemaphoreType.DMA,
    ],
)


def _edge_kernel(os_ref, oq_ref, ec_ref,
                 wr1_ref, br1_ref, wr2_ref, br2_ref,
                 w00_ref, w10_ref, w01_ref, w11_ref, summ_ref,
                 m0_ref, m1_ref, s_ref):
    osv = os_ref[...]
    f0 = osv[:, :32]
    f1 = osv[:, 32:80]
    q = oq_ref[...]
    ec = ec_ref[...]

    dot = lambda a, b: jnp.dot(a, b, preferred_element_type=f32)

    rad = ec
    hwr = jnp.maximum(dot(rad, wr1_ref[...]) + br1_ref[...], 0.0)
    w = dot(hwr, wr2_ref[...]) + br2_ref[...]
    w0 = w[:, :C0]
    w1 = w[:, C0:C0 + C1]

    d0 = ec[:, 5:6]
    d1 = ec[:, 6:7]
    d2 = ec[:, 7:8]
    f1a = f1[:, 0:16]
    f1b = f1[:, 16:32]
    f1c = f1[:, 32:48]
    bf = lambda x: x.astype(jnp.bfloat16).astype(f32)
    f1d = bf(f1a) * bf(d0) + bf(f1b) * bf(d1) + bf(f1c) * bf(d2)

    m0 = (dot(f0, w00_ref[...]) + dot(f1d, w10_ref[...])) * w0
    g = dot(f0, w01_ref[...])
    w11 = w11_ref[...]
    m1a = (dot(f1a, w11) + g * d0) * w1
    m1b = (dot(f1b, w11) + g * d1) * w1
    m1c = (dot(f1c, w11) + g * d2) * w1

    qk = q * m0
    s = jnp.dot(qk, summ_ref[...], preferred_element_type=f32,
                precision=jax.lax.Precision.HIGHEST) \
        * np.float32(1.0 / np.sqrt(HD))

    m0_ref[...] = m0
    m1_ref[...] = jnp.concatenate([m1a, m1b, m1c], axis=1)
    s_ref[...] = s


def _run_edge_layer(osv, oqv, ec, lp, summ):
    wr1 = jnp.zeros((8, RH), f32).at[:5, :].set(lp["Wr1"])
    wr2 = lp["Wr2"][:, :C0 + C1]
    br2 = lp["br2"][:C0 + C1]

    grid = (E // BE,)
    eb = lambda w: pl.BlockSpec((BE, w), lambda i: (i, 0))
    full = lambda a: pl.BlockSpec(a.shape, lambda i: (0,) * a.ndim)
    args = (osv, oqv, ec, wr1, lp["br1"], wr2, br2, lp["W00"], lp["W10"],
            lp["W01"], lp["W11"], summ)
    return pl.pallas_call(
        _edge_kernel,
        grid=grid,
        in_specs=[eb(80), eb(32), eb(8)] + [full(a) for a in args[3:]],
        out_specs=[eb(32), eb(48), eb(8)],
        out_shape=[
            jax.ShapeDtypeStruct((E, C0), f32),
            jax.ShapeDtypeStruct((E, 3 * C1), f32),
            jax.ShapeDtypeStruct((E, N_HEADS), f32),
        ],
    )(*args)


def kernel(f, pos, edge_attr, targets, edge_index, params):
    src = edge_index[0]
    dst = edge_index[1]

    posp = jnp.zeros((N, 8), f32).at[:, :3].set(pos)
    posS8, posD8 = _g_pos_call(posp, posp, src, dst)
    rel = posD8[:, :3] - posS8[:, :3]
    r = jnp.sqrt(jnp.sum(rel * rel, axis=-1, keepdims=True))
    dirv = rel / (r + 1e-8)
    ec = jnp.concatenate([r, edge_attr, dirv], axis=1)

    idx8 = (dst[:, None] * 8 + jnp.arange(8, dtype=i32)[None, :]).reshape(-1)
    dst2d = dst.reshape(E // 125, 125)
    summ = jnp.repeat(jnp.eye(N_HEADS, dtype=f32), HD, axis=0)

    h1_in = f[:, 1:4, 0]
    h1 = (h1_in[:, :, None] * params["Win1"][0][None, None, :]).reshape(N, 48)
    h0 = jnp.zeros((N, C0), f32)

    for lp in params["layers"]:
        tabS = jnp.concatenate([h0, h1], axis=1)
        tabQ = h0 @ lp["Wq0"]

        osv, oqv = _g_call(tabS, tabQ, src, dst)
        m0, m1, s = _run_edge_layer(osv, oqv, ec, lp, summ)

        sflat = s.reshape(-1)
        part = _a_call(idx8, sflat)
        ex = _b1_call(idx8, sflat, part)
        acc = _b2_call(ex, m0.reshape(-1), m1.reshape(-1), dst2d)

        accsum = acc[0] + acc[1]
        sum0 = accsum[:, :32]
        sum1 = accsum[:, 32:80]
        den = accsum[:, 80:88]
        den0 = jnp.repeat(den, HD, axis=1) + 1e-9
        den1 = jnp.tile(jnp.repeat(den, C1 // N_HEADS, axis=1), (1, 3)) + 1e-9
        h0 = h0 + sum0 / den0
        h1 = h1 + sum1 / den1

        n0 = jnp.abs(h0)
        h0 = jax.nn.relu(n0 @ lp["Wn0"] + lp["bn0"]) * jnp.sign(h0)
        h1v = h1.reshape(N, 3, C1)
        n1 = jnp.sqrt(jnp.sum(h1v * h1v, axis=1)) + 1e-8
        mult = jax.nn.relu(n1 @ lp["Wn1"] + lp["bn1"]) / n1
        h1 = (h1v * mult[:, None, :]).reshape(N, 48)

    fp = params["final"]
    rad_in = ec[:, :5]
    wf = jax.nn.relu(rad_in @ fp["Wr1"] + fp["br1"]) @ fp["Wr2"] + fp["br2"]
    h1cm = jnp.transpose(h1.reshape(N, 3, C1), (0, 2, 1))
    h1s48, _unused = _g_fin_call(h1cm.reshape(N, 48), posp, src, dst)
    me = jnp.einsum('ecd,ec->ed', h1s48.reshape(E, C1, 3), wf)
    out = jax.ops.segment_sum(me, dst, num_segments=N) \
        + jnp.einsum('ncd,c->nd', h1cm, fp["self_w"])
    vec = out[None, :, :]
    loss = jnp.mean(jnp.sqrt(jnp.sum((vec - targets) ** 2, axis=-1) + 1e-5),
                    axis=-1)
    return vec, loss
